# Initial kernel scaffold; baseline (speedup 1.0000x reference)
import jax
import jax.numpy as jnp
from jax.experimental import pallas as pl


def kernel(proposalN, x):
    raise NotImplementedError("write your pallas kernel here")


# speedup vs baseline: 8.2487x; 8.2487x over previous
import functools

import numpy as np
import jax
import jax.numpy as jnp
from jax import lax
from jax.experimental import pallas as pl
from jax.experimental.pallas import tpu as pltpu
from jax.experimental.pallas import tpu_sc as plsc

_STRIDE = 32
_SIZE = 14
_RATIOS = [[4, 4], [3, 5], [5, 3], [6, 6], [5, 7], [7, 5], [8, 8],
           [6, 10], [10, 6], [7, 9], [9, 7], [7, 10], [10, 7]]
_GROUPS = [(0, 3), (3, 6), (6, 13)]
_NSEL = [2, 3, 2]
_IOU_THR = 0.25
_PADW = 384
_NCHUNK = _PADW // 16


def _build_tables():
    w = np.zeros((_SIZE * _SIZE, 3 * _PADW), np.float32)
    coords = np.zeros((3 * 4, _PADW), np.float32)
    gsizes, glo = [], []
    goff = 0
    for g, (r0, r1) in enumerate(_GROUPS):
        j = 0
        glo.append(goff)
        for ri in range(r0, r1):
            kh, kw = _RATIOS[ri]
            nrows, ncols = _SIZE - kh + 1, _SIZE - kw + 1
            inv = 1.0 / float(kh * kw)
            for xi in range(nrows):
                for yi in range(ncols):
                    col = g * _PADW + j
                    for a in range(kh):
                        for b in range(kw):
                            w[(xi + a) * _SIZE + (yi + b), col] = inv
                    xl = xi * _STRIDE - 1
                    yl = yi * _STRIDE - 1
                    coords[g * 4 + 0, j] = max(xl, 0)
                    coords[g * 4 + 1, j] = max(yl, 0)
                    coords[g * 4 + 2, j] = xl + kh * _STRIDE
                    coords[g * 4 + 3, j] = yl + kw * _STRIDE
                    j += 1
        gsizes.append(j)
        goff += j
    return w, coords, gsizes, glo


_W_NP, _COORDS_NP, _GSIZES, _GLO = _build_tables()


def _score_body(x_ref, w_ref, o_ref):
    fm = jnp.sum(x_ref[...], axis=1)
    o_ref[...] = lax.dot(fm, w_ref[...],
                         precision=lax.Precision.HIGHEST,
                         preferred_element_type=jnp.float32)


def _scores_tc(x2, w):
    return pl.pallas_call(
        _score_body,
        out_shape=jax.ShapeDtypeStruct((x2.shape[0], 3 * _PADW), jnp.float32),
    )(x2, w)


def _nms_sc(scores_p, coords):
    info = plsc.get_sparse_core_info()
    nc = info.num_cores
    mesh = plsc.VectorSubcoreMesh(core_axis_name="c", subcore_axis_name="s")
    neg = jnp.float32(-jnp.inf)

    @functools.partial(
        pl.kernel,
        out_type=jax.ShapeDtypeStruct((24, 16), jnp.int32),
        mesh=mesh,
        compiler_params=pltpu.CompilerParams(needs_layout_passes=False),
        scratch_types=[
            pltpu.VMEM((_PADW,), jnp.float32),
            pltpu.VMEM((4, _PADW), jnp.float32),
            pltpu.VMEM((16,), jnp.int32),
            pltpu.VMEM((16,), jnp.float32),
            pltpu.VMEM((16,), jnp.int32),
        ],
    )
    def k(scores_hbm, coords_hbm, out_hbm, ms_v, cv, oi_v, tf_v, ti_v):
        wid = lax.axis_index("s") * nc + lax.axis_index("c")

        @pl.when(wid < 24)
        def _():
            b = wid // 3
            g = wid - 3 * b
            ngw = jnp.where(g == 0, _GSIZES[0],
                            jnp.where(g == 1, _GSIZES[1], _GSIZES[2]))
            lo = jnp.where(g == 0, _GLO[0],
                           jnp.where(g == 1, _GLO[1], _GLO[2]))
            pltpu.sync_copy(scores_hbm.at[b, pl.ds(g * _PADW, _PADW)], ms_v)
            pltpu.sync_copy(coords_hbm.at[pl.ds(g * 4, 4)], cv)
            iota = lax.broadcasted_iota(jnp.int32, (16,), 0)

            def initbody(ci, _):
                st = ci * 16
                v = ms_v[pl.ds(st, 16)]
                ms_v[pl.ds(st, 16)] = jnp.where(iota + st < ngw, v, neg)
                return 0

            lax.fori_loop(0, _NCHUNK, initbody, 0)

            def allmax_f(v):
                for sh in (8, 4, 2, 1):
                    tf_v[...] = v
                    v = jnp.maximum(v, plsc.load_gather(tf_v, [iota ^ sh]))
                return v

            def allmax_i(v):
                for sh in (8, 4, 2, 1):
                    ti_v[...] = v
                    v = jnp.maximum(v, plsc.load_gather(ti_v, [iota ^ sh]))
                return v

            oivec = jnp.zeros((16,), jnp.int32)
            lastv = jnp.zeros((16,), jnp.int32)
            for t in range(3):
                def maxbody(ci, carry):
                    mv, mi = carry
                    st = ci * 16
                    v = ms_v[pl.ds(st, 16)]
                    cond = v >= mv
                    return (jnp.where(cond, v, mv),
                            jnp.where(cond, iota + st, mi))

                mv, mi = lax.fori_loop(
                    0, _NCHUNK, maxbody,
                    (jnp.full((16,), neg, jnp.float32),
                     jnp.zeros((16,), jnp.int32)))
                m = allmax_f(mv)
                curv = allmax_i(jnp.where(mv == m, mi, -1))
                curv = jnp.where(m != neg, curv, lastv)
                lastv = curv

                cxl = plsc.load_gather(cv, [jnp.full((16,), 0, jnp.int32), curv])
                cyl = plsc.load_gather(cv, [jnp.full((16,), 1, jnp.int32), curv])
                cxr = plsc.load_gather(cv, [jnp.full((16,), 2, jnp.int32), curv])
                cyr = plsc.load_gather(cv, [jnp.full((16,), 3, jnp.int32), curv])
                areac = (cxr - cxl + 1.0) * (cyr - cyl + 1.0)

                oivec = jnp.where(iota == t, curv + lo, oivec)

                def supbody(ci, _):
                    st = ci * 16
                    xlv = cv[0, pl.ds(st, 16)]
                    ylv = cv[1, pl.ds(st, 16)]
                    xrv = cv[2, pl.ds(st, 16)]
                    yrv = cv[3, pl.ds(st, 16)]
                    l0 = jnp.minimum(xrv, cxr) - jnp.maximum(xlv, cxl) + 1.0
                    l1 = jnp.minimum(yrv, cyr) - jnp.maximum(ylv, cyl) + 1.0
                    inter = jnp.where((l0 < 0.0) | (l1 < 0.0), 0.0, l0 * l1)
                    areav = (xrv - xlv + 1.0) * (yrv - ylv + 1.0)
                    union = areav + areac - inter
                    keep = (inter <= _IOU_THR * union) & (iota + st != curv)
                    vv = ms_v[pl.ds(st, 16)]
                    ms_v[pl.ds(st, 16)] = jnp.where(keep, vv, neg)
                    return 0

                lax.fori_loop(0, _NCHUNK, supbody, 0)

            oi_v[...] = oivec
            pltpu.sync_copy(oi_v, out_hbm.at[wid])

    return k(scores_p, coords)


def kernel(proposalN, x):
    batch = x.shape[0]
    x2 = x.reshape(batch, x.shape[1], _SIZE * _SIZE)
    sp = _scores_tc(x2, jnp.asarray(_W_NP))
    window_scores = jnp.concatenate(
        [sp[:, _PADW * g:_PADW * g + _GSIZES[g]] for g in range(3)], axis=1)
    idx24 = _nms_sc(sp, jnp.asarray(_COORDS_NP))
    r = idx24.reshape(batch, 3, 16)
    idx = jnp.concatenate([r[:, g, :_NSEL[g]] for g in range(3)], axis=1)
    idx = idx + (proposalN - sum(_NSEL))
    gathered = jnp.take_along_axis(window_scores, idx, axis=1)
    return (idx, gathered, window_scores)
